# baseline (speedup 1.0000x reference)
import functools

import jax
import jax.numpy as jnp
from jax import lax
from jax.experimental import pallas as pl
from jax.experimental.pallas import tpu as pltpu
from jax.experimental.pallas import tpu_sc as plsc

_BB = 32
_HEAD = 128


_SQUARES = 64
_PER_W = 8


def _sc_gather(re_hbm, ce_hbm, rows_hbm, cols_hbm, posr_hbm, posc_hbm,
               idx_v, buf_v, sem):
    wid = lax.axis_index("s")

    @pl.when(wid == 0)
    def _():
        pltpu.sync_copy(rows_hbm, idx_v)
        pltpu.async_copy(re_hbm.at[idx_v], buf_v, sem).wait()
        pltpu.sync_copy(buf_v, posr_hbm)

    @pl.when(wid == 1)
    def _():
        pltpu.sync_copy(cols_hbm, idx_v)
        pltpu.async_copy(ce_hbm.at[idx_v], buf_v, sem).wait()
        pltpu.sync_copy(buf_v, posc_hbm)


def _sc_positions(row_embed, col_embed, rows, cols):
    D = row_embed.shape[1]
    mesh = plsc.VectorSubcoreMesh(
        core_axis_name="c", subcore_axis_name="s", num_cores=1, num_subcores=2
    )
    fn = functools.partial(
        pl.kernel,
        out_type=(
            jax.ShapeDtypeStruct((_SQUARES, D), jnp.float32),
            jax.ShapeDtypeStruct((_SQUARES, D), jnp.float32),
        ),
        mesh=mesh,
        scratch_types=[
            pltpu.VMEM((_SQUARES,), jnp.int32),
            pltpu.VMEM((_SQUARES, D), jnp.float32),
            pltpu.SemaphoreType.DMA,
        ],
    )(_sc_gather)
    return fn(row_embed, col_embed, rows, cols)


def _tc_head_body(rows_sref, cols_sref, tok_ref, re_ref, ce_ref, out_ref,
                  pos_ref):
    i = pl.program_id(0)

    @pl.when(i == 0)
    def _():
        def loop_body(s, _):
            pos_ref[s, :] = re_ref[rows_sref[s], :] + ce_ref[cols_sref[s], :]
            return ()

        jax.lax.fori_loop(0, pos_ref.shape[0], loop_body, ())

    out_ref[...] = tok_ref[...] + pos_ref[...][None, :, :]


def _tc_tail_body(posr_ref, posc_ref, tok_ref, head_ref, out_ref, pos_ref):
    del head_ref
    i = pl.program_id(0)

    @pl.when(i == 0)
    def _():
        pos_ref[...] = posr_ref[...] + posc_ref[...]

    out_ref[...] = tok_ref[...] + pos_ref[...][None, :, :]


def kernel(tokens, row_embed, col_embed, rows, cols):
    B, S, D = tokens.shape
    rows32 = rows.astype(jnp.int32)
    cols32 = cols.astype(jnp.int32)
    head_blocks = _HEAD // _BB
    tail_blocks = (B - _HEAD) // _BB

    posr, posc = _sc_positions(row_embed, col_embed, rows32, cols32)

    head = pl.pallas_call(
        _tc_head_body,
        grid_spec=pltpu.PrefetchScalarGridSpec(
            num_scalar_prefetch=2,
            grid=(head_blocks,),
            in_specs=[
                pl.BlockSpec((_BB, S, D), lambda i, r, c: (i, 0, 0)),
                pl.BlockSpec((8, D), lambda i, r, c: (0, 0)),
                pl.BlockSpec((8, D), lambda i, r, c: (0, 0)),
            ],
            out_specs=pl.BlockSpec((_BB, S, D), lambda i, r, c: (i, 0, 0)),
            scratch_shapes=[pltpu.VMEM((S, D), jnp.float32)],
        ),
        out_shape=jax.ShapeDtypeStruct((B, S, D), tokens.dtype),
        compiler_params=pltpu.CompilerParams(
            dimension_semantics=("arbitrary",),
        ),
    )(rows32, cols32, tokens, row_embed, col_embed)

    off = head_blocks
    out = pl.pallas_call(
        _tc_tail_body,
        grid=(tail_blocks,),
        in_specs=[
            pl.BlockSpec((S, D), lambda i: (0, 0)),
            pl.BlockSpec((S, D), lambda i: (0, 0)),
            pl.BlockSpec((_BB, S, D), lambda i: (i + off, 0, 0)),
            pl.BlockSpec(memory_space=pl.ANY),
        ],
        out_specs=pl.BlockSpec((_BB, S, D), lambda i: (i + off, 0, 0)),
        out_shape=jax.ShapeDtypeStruct((B, S, D), tokens.dtype),
        scratch_shapes=[pltpu.VMEM((S, D), jnp.float32)],
        input_output_aliases={3: 0},
        compiler_params=pltpu.CompilerParams(
            dimension_semantics=("arbitrary",),
        ),
    )(posr, posc, tokens, head)
    return out


# speedup vs baseline: 1.0011x; 1.0011x over previous
import functools

import jax
import jax.numpy as jnp
from jax import lax
from jax.experimental import pallas as pl
from jax.experimental.pallas import tpu as pltpu
from jax.experimental.pallas import tpu_sc as plsc

_BB = 32
_HEAD = 256


_SQUARES = 64
_PER_W = 8


def _sc_gather(re_hbm, ce_hbm, rows_hbm, cols_hbm, posr_hbm, posc_hbm,
               idx_v, buf_v, sem):
    wid = lax.axis_index("s")
    nw_half = _SQUARES // _PER_W

    @pl.when(wid < nw_half)
    def _():
        base = wid * _PER_W
        pltpu.sync_copy(rows_hbm.at[pl.ds(base, _PER_W)], idx_v)
        pltpu.async_copy(re_hbm.at[idx_v], buf_v, sem).wait()
        pltpu.sync_copy(buf_v, posr_hbm.at[pl.ds(base, _PER_W)])

    @pl.when((wid >= nw_half) & (wid < 2 * nw_half))
    def _():
        base = (wid - nw_half) * _PER_W
        pltpu.sync_copy(cols_hbm.at[pl.ds(base, _PER_W)], idx_v)
        pltpu.async_copy(ce_hbm.at[idx_v], buf_v, sem).wait()
        pltpu.sync_copy(buf_v, posc_hbm.at[pl.ds(base, _PER_W)])


def _sc_positions(row_embed, col_embed, rows, cols):
    D = row_embed.shape[1]
    mesh = plsc.VectorSubcoreMesh(
        core_axis_name="c", subcore_axis_name="s", num_cores=1
    )
    fn = functools.partial(
        pl.kernel,
        out_type=(
            jax.ShapeDtypeStruct((_SQUARES, D), jnp.float32),
            jax.ShapeDtypeStruct((_SQUARES, D), jnp.float32),
        ),
        mesh=mesh,
        scratch_types=[
            pltpu.VMEM((_PER_W,), jnp.int32),
            pltpu.VMEM((_PER_W, D), jnp.float32),
            pltpu.SemaphoreType.DMA,
        ],
    )(_sc_gather)
    return fn(row_embed, col_embed, rows, cols)


def _tc_head_body(rows_sref, cols_sref, tok_ref, re_ref, ce_ref, out_ref,
                  pos_ref):
    i = pl.program_id(0)

    @pl.when(i == 0)
    def _():
        def loop_body(s, _):
            pos_ref[s, :] = re_ref[rows_sref[s], :] + ce_ref[cols_sref[s], :]
            return ()

        jax.lax.fori_loop(0, pos_ref.shape[0], loop_body, ())

    out_ref[...] = tok_ref[...] + pos_ref[...][None, :, :]


def _tc_tail_body(posr_ref, posc_ref, tok_ref, head_ref, out_ref, pos_ref):
    del head_ref
    i = pl.program_id(0)

    @pl.when(i == 0)
    def _():
        pos_ref[...] = posr_ref[...] + posc_ref[...]

    out_ref[...] = tok_ref[...] + pos_ref[...][None, :, :]


def kernel(tokens, row_embed, col_embed, rows, cols):
    B, S, D = tokens.shape
    rows32 = rows.astype(jnp.int32)
    cols32 = cols.astype(jnp.int32)
    head_blocks = _HEAD // _BB
    tail_blocks = (B - _HEAD) // _BB

    posr, posc = _sc_positions(row_embed, col_embed, rows32, cols32)

    head = pl.pallas_call(
        _tc_head_body,
        grid_spec=pltpu.PrefetchScalarGridSpec(
            num_scalar_prefetch=2,
            grid=(head_blocks,),
            in_specs=[
                pl.BlockSpec((_BB, S, D), lambda i, r, c: (i, 0, 0)),
                pl.BlockSpec((8, D), lambda i, r, c: (0, 0)),
                pl.BlockSpec((8, D), lambda i, r, c: (0, 0)),
            ],
            out_specs=pl.BlockSpec((_BB, S, D), lambda i, r, c: (i, 0, 0)),
            scratch_shapes=[pltpu.VMEM((S, D), jnp.float32)],
        ),
        out_shape=jax.ShapeDtypeStruct((B, S, D), tokens.dtype),
        compiler_params=pltpu.CompilerParams(
            dimension_semantics=("arbitrary",),
        ),
    )(rows32, cols32, tokens, row_embed, col_embed)

    off = head_blocks
    out = pl.pallas_call(
        _tc_tail_body,
        grid=(tail_blocks,),
        in_specs=[
            pl.BlockSpec((S, D), lambda i: (0, 0)),
            pl.BlockSpec((S, D), lambda i: (0, 0)),
            pl.BlockSpec((_BB, S, D), lambda i: (i + off, 0, 0)),
            pl.BlockSpec(memory_space=pl.ANY),
        ],
        out_specs=pl.BlockSpec((_BB, S, D), lambda i: (i + off, 0, 0)),
        out_shape=jax.ShapeDtypeStruct((B, S, D), tokens.dtype),
        scratch_shapes=[pltpu.VMEM((S, D), jnp.float32)],
        input_output_aliases={3: 0},
        compiler_params=pltpu.CompilerParams(
            dimension_semantics=("arbitrary",),
        ),
    )(posr, posc, tokens, head)
    return out
